# Initial kernel scaffold; baseline (speedup 1.0000x reference)
import jax
import jax.numpy as jnp
from jax.experimental import pallas as pl


def kernel(params, tokens, edge):
    raise NotImplementedError("write your pallas kernel here")


# speedup vs baseline: 1.1060x; 1.1060x over previous
import functools

import jax
import jax.numpy as jnp
from jax import lax
from jax.experimental import pallas as pl
from jax.experimental.pallas import tpu as pltpu
from jax.experimental.pallas import tpu_sc as plsc

_B, _S, _D, _V = 128, 77, 256, 50000
_SP = 80
_G = 8
_ROWS = _G * _SP
_N = _B * _SP
_IDX_CHUNK = 80


def _sc_gather(table, idx):
    info = plsc.get_sparse_core_info()
    nw = info.num_cores * info.num_subcores
    b_per_w = _N // nw
    n_chunks = b_per_w // _IDX_CHUNK

    mesh = plsc.VectorSubcoreMesh(core_axis_name="c", subcore_axis_name="s")

    @functools.partial(
        pl.kernel,
        out_type=jax.ShapeDtypeStruct((_N, _D), jnp.float32),
        mesh=mesh,
        scratch_types=[
            pltpu.VMEM((n_chunks, _IDX_CHUNK), jnp.int32),
            pltpu.VMEM((b_per_w, _D), jnp.float32),
            pltpu.SemaphoreType.DMA,
        ],
    )
    def k(table_hbm, idx_hbm, out_hbm, idx_v, rows_v, sem):
        wid = lax.axis_index("s") * info.num_cores + lax.axis_index("c")
        base = wid * b_per_w
        pltpu.sync_copy(idx_hbm.at[wid], idx_v)
        copies = []
        for c in range(n_chunks):
            copies.append(pltpu.async_copy(
                table_hbm.at[idx_v.at[c]],
                rows_v.at[pl.ds(c * _IDX_CHUNK, _IDX_CHUNK)], sem))
        for cp in copies:
            cp.wait()
        pltpu.sync_copy(rows_v, out_hbm.at[pl.ds(base, b_per_w)])

    return k(table, idx.reshape(nw, n_chunks, _IDX_CHUNK))


def _norm(u):
    return jnp.sqrt(jnp.sum(u * u, axis=-1, keepdims=True))


def _expmap0(u):
    n = jnp.maximum(_norm(u), 1e-15)
    return jnp.tanh(n) * u / n


def _logmap0(y):
    n = jnp.maximum(_norm(y), 1e-15)
    nc = jnp.minimum(n, 1.0 - 1e-7)
    return 0.5 * jnp.log((1.0 + nc) / (1.0 - nc)) * y / n


def _nt(a, w):
    return lax.dot_general(a, w, (((1,), (1,)), ((), ())),
                           preferred_element_type=jnp.float32)


def _tc_body(x_ref, edge_ref, pos_ref,
             w0, w1, w2, w3, w4, pb_ref,
             wrel, wroot, gb_ref, out_ref):
    x = x_ref[...] + pos_ref[...]
    ws = (w0, w1, w2, w3, w4)
    for i in range(5):
        x = _nt(x, ws[i][...]) + pb_ref[i, :][None, :]
        if i < 4:
            x = x * jax.nn.sigmoid(x)
    xt = _logmap0(_expmap0(x))
    msgs = []
    for s in range(_G):
        a = (edge_ref[s] != 0).astype(jnp.float32)
        xs = xt[s * _SP:(s + 1) * _SP]
        msgs.append(lax.dot_general(a, xs, (((0,), (0,)), ((), ())),
                                    preferred_element_type=jnp.float32))
    msg = jnp.concatenate(msgs, axis=0)
    out_t = _nt(msg, wrel[...]) + _nt(xt, wroot[...]) + gb_ref[0, :][None, :]
    t = _logmap0(_expmap0(out_t))
    t = jnp.where(t >= 0, t, 0.01 * t)
    out_ref[...] = _logmap0(_expmap0(t))


def _tc_dense(x, edge_pad, pos_tiled, proj_w, proj_b, wrel, wroot, gcn_b):
    n_blocks = _N // _ROWS
    row_spec = pl.BlockSpec((_ROWS, _D), lambda i: (i, 0))
    const2 = pl.BlockSpec((_D, _D), lambda i: (0, 0))
    grid_spec = pl.GridSpec(
        grid=(n_blocks,),
        in_specs=[
            row_spec,
            pl.BlockSpec((_G, _SP, _SP), lambda i: (i, 0, 0)),
            row_spec,
            const2, const2, const2, const2, const2,
            pl.BlockSpec((5, _D), lambda i: (0, 0)),
            const2, const2,
            pl.BlockSpec((1, _D), lambda i: (0, 0)),
        ],
        out_specs=row_spec,
    )
    return pl.pallas_call(
        _tc_body,
        grid_spec=grid_spec,
        out_shape=jax.ShapeDtypeStruct((_N, _D), jnp.float32),
    )(x, edge_pad, pos_tiled,
      proj_w[0], proj_w[1], proj_w[2], proj_w[3], proj_w[4],
      jnp.stack(proj_b), wrel, wroot, gcn_b[None, :])


def kernel(params, tokens, edge):
    tokens = tokens.astype(jnp.int32)
    idx = jnp.pad(tokens, ((0, 0), (0, _SP - _S))).reshape(_N)
    x = _sc_gather(params["token_table"], idx)

    pos_pad = jnp.pad(params["pos_table"], ((0, _SP - _S), (0, 0)))
    pos_tiled = jnp.broadcast_to(
        pos_pad[None], (_B, _SP, _D)).reshape(_N, _D)

    edge_pad = jnp.pad(edge.astype(jnp.int32),
                       ((0, 0), (0, _SP - _S), (0, _SP - _S)))

    y = _tc_dense(x, edge_pad, pos_tiled,
                  params["proj_W"], params["proj_b"],
                  params["gcn_Wrel"][3], params["gcn_Wroot"][3],
                  params["gcn_b"][3])
    return y.reshape(_B, _SP, _D)[:, :_S, :]


# speedup vs baseline: 1.2086x; 1.0927x over previous
import functools

import jax
import jax.numpy as jnp
from jax import lax
from jax.experimental import pallas as pl
from jax.experimental.pallas import tpu as pltpu
from jax.experimental.pallas import tpu_sc as plsc

_B, _S, _D, _V = 128, 77, 256, 50000
_SP = 80
_G = 8
_ROWS = _G * _SP
_N = _B * _SP
_IDX_CHUNK = 80


def _sc_gather(table, idx):
    info = plsc.get_sparse_core_info()
    nw = info.num_cores * info.num_subcores
    b_per_w = _N // nw
    n_chunks = b_per_w // _IDX_CHUNK

    mesh = plsc.VectorSubcoreMesh(core_axis_name="c", subcore_axis_name="s")

    @functools.partial(
        pl.kernel,
        out_type=jax.ShapeDtypeStruct((_N, _D), jnp.float32),
        mesh=mesh,
        scratch_types=[
            pltpu.VMEM((n_chunks, _IDX_CHUNK), jnp.int32),
            pltpu.VMEM((b_per_w, _D), jnp.float32),
            pltpu.SemaphoreType.DMA,
        ],
    )
    def k(table_hbm, idx_hbm, out_hbm, idx_v, rows_v, sem):
        wid = lax.axis_index("s") * info.num_cores + lax.axis_index("c")
        base = wid * b_per_w
        pltpu.sync_copy(idx_hbm.at[wid], idx_v)
        copies = []
        for c in range(n_chunks):
            copies.append(pltpu.async_copy(
                table_hbm.at[idx_v.at[c]],
                rows_v.at[pl.ds(c * _IDX_CHUNK, _IDX_CHUNK)], sem))
        for cp in copies:
            cp.wait()
        pltpu.sync_copy(rows_v, out_hbm.at[pl.ds(base, b_per_w)])

    return k(table, idx.reshape(nw, n_chunks, _IDX_CHUNK))


def _cap(u):
    n = jnp.maximum(jnp.sqrt(jnp.sum(u * u, axis=-1, keepdims=True)), 1e-15)
    thc = jnp.minimum(jnp.tanh(n), 1.0 - 1e-7)
    s = 0.5 * jnp.log((1.0 + thc) / (1.0 - thc)) / n
    return u * s


def _nt(a, w):
    return lax.dot_general(a, w, (((1,), (1,)), ((), ())),
                           preferred_element_type=jnp.float32)


def _tc_body(x_ref, edge_ref, pos_ref,
             w0, w1, w2, w3, w4, pb_ref,
             wrel, wroot, gb_ref, out_ref):
    x = x_ref[...] + pos_ref[...]
    ws = (w0, w1, w2, w3, w4)
    for i in range(5):
        x = _nt(x, ws[i][...]) + pb_ref[i, :][None, :]
        if i < 4:
            x = x * (0.5 + 0.5 * jnp.tanh(0.5 * x))
    xt = _cap(x)
    msgs = []
    for s in range(_G):
        a = (edge_ref[s] != 0).astype(jnp.float32)
        xs = xt[s * _SP:(s + 1) * _SP]
        msgs.append(lax.dot_general(a, xs, (((0,), (0,)), ((), ())),
                                    preferred_element_type=jnp.float32))
    msg = jnp.concatenate(msgs, axis=0)
    out_t = _nt(msg, wrel[...]) + _nt(xt, wroot[...]) + gb_ref[0, :][None, :]
    t = _cap(out_t)
    t = jnp.where(t >= 0, t, 0.01 * t)
    out_ref[...] = _cap(t)


def _tc_dense(x, edge_pad, pos_tiled, proj_w, proj_b, wrel, wroot, gcn_b):
    n_blocks = _N // _ROWS
    row_spec = pl.BlockSpec((_ROWS, _D), lambda i: (i, 0))
    const2 = pl.BlockSpec((_D, _D), lambda i: (0, 0))
    grid_spec = pl.GridSpec(
        grid=(n_blocks,),
        in_specs=[
            row_spec,
            pl.BlockSpec((_G, _SP, _SP), lambda i: (i, 0, 0)),
            row_spec,
            const2, const2, const2, const2, const2,
            pl.BlockSpec((5, _D), lambda i: (0, 0)),
            const2, const2,
            pl.BlockSpec((1, _D), lambda i: (0, 0)),
        ],
        out_specs=row_spec,
    )
    return pl.pallas_call(
        _tc_body,
        grid_spec=grid_spec,
        out_shape=jax.ShapeDtypeStruct((_N, _D), jnp.float32),
    )(x, edge_pad, pos_tiled,
      proj_w[0], proj_w[1], proj_w[2], proj_w[3], proj_w[4],
      jnp.stack(proj_b), wrel, wroot, gcn_b[None, :])


def kernel(params, tokens, edge):
    tokens = tokens.astype(jnp.int32)
    idx = jnp.pad(tokens, ((0, 0), (0, _SP - _S))).reshape(_N)
    x = _sc_gather(params["token_table"], idx)

    pos_pad = jnp.pad(params["pos_table"], ((0, _SP - _S), (0, 0)))
    pos_tiled = jnp.broadcast_to(
        pos_pad[None], (_B, _SP, _D)).reshape(_N, _D)

    edge_pad = jnp.pad(edge.astype(jnp.int32),
                       ((0, 0), (0, _SP - _S), (0, _SP - _S)))

    y = _tc_dense(x, edge_pad, pos_tiled,
                  params["proj_W"], params["proj_b"],
                  params["gcn_Wrel"][3], params["gcn_Wroot"][3],
                  params["gcn_b"][3])
    return y.reshape(_B, _SP, _D)[:, :_S, :]
